# baseline (speedup 1.0000x reference)
import functools
import math

import jax
import jax.numpy as jnp
from jax import lax
from jax.experimental import pallas as pl
from jax.experimental.pallas import tpu as pltpu
from jax.experimental.pallas import tpu_sc as plsc

N = 10000
E = 320000
D = 128
ED = 16
BN_EPS = 1e-5

NC = 2
NS = 16
LANE = 16

G = 80
ROWS = E // G
RPS = ROWS // NC
RPT = RPS // NS
NZR = N // NS
NBUF = 2
TRIPS = RPT // NBUF


def _sc_aggregate(x, idx_cat, ep):
  mesh = plsc.VectorSubcoreMesh(core_axis_name="c", subcore_axis_name="s")

  @functools.partial(
      pl.kernel,
      out_type=jax.ShapeDtypeStruct((NC, N, D), jnp.float32),
      mesh=mesh,
      compiler_params=pltpu.CompilerParams(use_tc_tiling_on_sc=False),
      scratch_types=[
          pltpu.VMEM_SHARED((N, D), jnp.float32),
          [pltpu.VMEM((1, 2, G), jnp.int32) for _ in range(NBUF)],
          [pltpu.VMEM((G, D), jnp.bfloat16) for _ in range(NBUF)],
          [pltpu.VMEM((G, D), jnp.float32) for _ in range(NBUF)],
          [pltpu.SemaphoreType.DMA for _ in range(NBUF)],
          [pltpu.SemaphoreType.DMA for _ in range(NBUF)],
          [pltpu.SemaphoreType.DMA for _ in range(NBUF)],
          [pltpu.SemaphoreType.DMA for _ in range(NBUF)],
      ],
  )
  def agg_kernel(x_hbm, idx_hbm, ep_hbm, out_hbm,
                 aggr_sh, idx_b, ep_b, rows_b, ix_sem, ep_sem, g_sem, sc_sem):
    c = lax.axis_index("c")
    s = lax.axis_index("s")
    row0 = c * RPS + s * RPT

    def _zrow(r, carry):
      for k in range(D // LANE):
        rows_b[0][r, pl.ds(k * LANE, LANE)] = jnp.zeros((LANE,), jnp.float32)
      return carry
    lax.fori_loop(0, G, _zrow, 0)
    z0 = s * NZR
    nfull = NZR // G
    for q in range(nfull):
      pltpu.sync_copy(rows_b[0], aggr_sh.at[pl.ds(z0 + q * G, G)])
    rem = NZR - nfull * G
    if rem:
      pltpu.sync_copy(rows_b[0].at[pl.ds(0, rem)],
                      aggr_sh.at[pl.ds(z0 + nfull * G, rem)])
    plsc.subcore_barrier()

    def stage1(ci, b):
      r = row0 + ci
      pltpu.async_copy(idx_hbm.at[pl.ds(r, 1)], idx_b[b], ix_sem[b])
      pltpu.async_copy(ep_hbm.at[pl.ds(r * G, G)], ep_b[b], ep_sem[b])

    def stage2(ci, b):
      r = row0 + ci
      pltpu.make_async_copy(idx_hbm.at[pl.ds(r, 1)], idx_b[b],
                            ix_sem[b]).wait()
      pltpu.async_copy(x_hbm.at[idx_b[b].at[0, 0]], rows_b[b], g_sem[b])

    def consume(ci, b):
      r = row0 + ci
      pltpu.make_async_copy(ep_hbm.at[pl.ds(r * G, G)], ep_b[b],
                            ep_sem[b]).wait()
      pltpu.make_async_copy(x_hbm.at[idx_b[b].at[0, 0]], rows_b[b],
                            g_sem[b]).wait()

      def _crow(rr, inner):
        for k in range(D // LANE):
          sl = pl.ds(k * LANE, LANE)
          rows_b[b][rr, sl] = jnp.maximum(
              rows_b[b][rr, sl] + ep_b[b][rr, sl].astype(jnp.float32), 0.0)
        return inner
      lax.fori_loop(0, G, _crow, 0)
      pltpu.async_copy(rows_b[b], aggr_sh.at[idx_b[b].at[0, 1]], sc_sem[b],
                       add=True)

    def wait_scatter(b):
      pltpu.make_async_copy(rows_b[b], aggr_sh.at[idx_b[b].at[0, 1]],
                            sc_sem[b]).wait()

    stage1(0, 0)
    stage1(1, 1)
    stage2(0, 0)

    def _pair(t, carry):
      for j in range(NBUF):
        ci = t * NBUF + j

        @pl.when(ci + 1 < RPT)
        def _():
          stage2(ci + 1, (j + 1) % NBUF)

        consume(ci, j)
        wait_scatter(j)

        @pl.when(ci + NBUF < RPT)
        def _():
          stage1(ci + NBUF, j)
      return carry
    lax.fori_loop(0, TRIPS, _pair, 0)
    for ci in range(TRIPS * NBUF, RPT):
      consume(ci, ci % NBUF)
      wait_scatter(ci % NBUF)

    plsc.subcore_barrier()
    pltpu.sync_copy(aggr_sh.at[pl.ds(z0, NZR)],
                    out_hbm.at[c, pl.ds(z0, NZR)])

  return agg_kernel(x, idx_cat, ep)


def _edge_proj(edge_attr, We, be):
  BE = 2000

  def body(ea_ref, we_ref, be_ref, out_ref):
    out_ref[...] = (jnp.dot(ea_ref[...], we_ref[...],
                            preferred_element_type=jnp.float32)
                    + be_ref[...]).astype(jnp.bfloat16)

  return pl.pallas_call(
      body,
      grid=(E // BE,),
      in_specs=[
          pl.BlockSpec((BE, ED), lambda i: (i, 0)),
          pl.BlockSpec((ED, D), lambda i: (0, 0)),
          pl.BlockSpec((1, D), lambda i: (0, 0)),
      ],
      out_specs=pl.BlockSpec((BE, D), lambda i: (i, 0)),
      out_shape=jax.ShapeDtypeStruct((E, D), jnp.bfloat16),
  )(edge_attr, We, be.reshape(1, D))


def _node_update(x, aggr, Wp, bp):
  BN = 1000

  def body(x_ref, a_ref, w_ref, b_ref, out_ref):
    y = x_ref[...] + a_ref[0] + a_ref[1]
    out_ref[...] = jnp.maximum(
        jnp.dot(y, w_ref[...], preferred_element_type=jnp.float32)
        + b_ref[...], 0.0)

  return pl.pallas_call(
      body,
      grid=(N // BN,),
      in_specs=[
          pl.BlockSpec((BN, D), lambda i: (i, 0)),
          pl.BlockSpec((NC, BN, D), lambda i: (0, i, 0)),
          pl.BlockSpec((D, D), lambda i: (0, 0)),
          pl.BlockSpec((1, D), lambda i: (0, 0)),
      ],
      out_specs=pl.BlockSpec((BN, D), lambda i: (i, 0)),
      out_shape=jax.ShapeDtypeStruct((N, D), jnp.float32),
  )(x, aggr, Wp, bp.reshape(1, D))


def kernel(x, edge_index, edge_attr,
           We0, be0, W0, b0, g0, bt0,
           We1, be1, W1, b1, g1, bt1,
           We2, be2, W2, b2, g2, bt2):
  scale = 1.0 / math.sqrt(1.0 + BN_EPS)
  src_rows = edge_index[0].reshape(ROWS, G)
  dst_rows = edge_index[1].reshape(ROWS, G)
  idx_cat = jnp.stack([src_rows, dst_rows], axis=1)

  eps = [_edge_proj(edge_attr, We, be)
         for We, be in ((We0, be0), (We1, be1), (We2, be2))]

  h = x
  for ep, W, b, g, bt in (
      (eps[0], W0, b0, g0, bt0),
      (eps[1], W1, b1, g1, bt1),
      (eps[2], W2, b2, g2, bt2)):
    aggr = _sc_aggregate(h, idx_cat, ep)
    gs = g * scale
    h = _node_update(h, aggr, W * gs[None, :], b * gs + bt)
  return h


# speedup vs baseline: 2.4453x; 2.4453x over previous
import functools
import math

import jax
import jax.numpy as jnp
from jax import lax
from jax.experimental import pallas as pl
from jax.experimental.pallas import tpu as pltpu
from jax.experimental.pallas import tpu_sc as plsc

N = 10000
E = 320000
D = 128
ED = 16
BN_EPS = 1e-5

NC = 2
NS = 16
LANE = 16

G = 80
ROWS = E // G
RPS = ROWS // NC
RPT = RPS // NS
NZR = N // NS
NBUF = 2
TRIPS = RPT // NBUF


def _sc_aggregate(x, idx_cat, ep):
  mesh = plsc.VectorSubcoreMesh(core_axis_name="c", subcore_axis_name="s")

  @functools.partial(
      pl.kernel,
      out_type=jax.ShapeDtypeStruct((NC, N, D), jnp.float32),
      mesh=mesh,
      compiler_params=pltpu.CompilerParams(use_tc_tiling_on_sc=False),
      scratch_types=[
          pltpu.VMEM_SHARED((N, D), jnp.float32),
          [pltpu.VMEM((1, 2, G), jnp.int32) for _ in range(NBUF)],
          [pltpu.VMEM((G, D), jnp.float32) for _ in range(NBUF)],
          [pltpu.VMEM((G, D), jnp.float32) for _ in range(NBUF)],
          [pltpu.SemaphoreType.DMA for _ in range(NBUF)],
          [pltpu.SemaphoreType.DMA for _ in range(NBUF)],
          [pltpu.SemaphoreType.DMA for _ in range(NBUF)],
          [pltpu.SemaphoreType.DMA for _ in range(NBUF)],
      ],
  )
  def agg_kernel(x_hbm, idx_hbm, ep_hbm, out_hbm,
                 aggr_sh, idx_b, ep_b, rows_b, ix_sem, ep_sem, g_sem, sc_sem):
    c = lax.axis_index("c")
    s = lax.axis_index("s")
    row0 = c * RPS + s * RPT

    def _zrow(r, carry):
      for k in range(D // LANE):
        rows_b[0][r, pl.ds(k * LANE, LANE)] = jnp.zeros((LANE,), jnp.float32)
      return carry
    lax.fori_loop(0, G, _zrow, 0)
    z0 = s * NZR
    nfull = NZR // G
    for q in range(nfull):
      pltpu.sync_copy(rows_b[0], aggr_sh.at[pl.ds(z0 + q * G, G)])
    rem = NZR - nfull * G
    if rem:
      pltpu.sync_copy(rows_b[0].at[pl.ds(0, rem)],
                      aggr_sh.at[pl.ds(z0 + nfull * G, rem)])
    plsc.subcore_barrier()

    def stage1(ci, b):
      r = row0 + ci
      pltpu.async_copy(idx_hbm.at[pl.ds(r, 1)], idx_b[b], ix_sem[b])
      pltpu.async_copy(ep_hbm.at[pl.ds(r * G, G)], ep_b[b], ep_sem[b])

    def stage2(ci, b):
      r = row0 + ci
      pltpu.make_async_copy(idx_hbm.at[pl.ds(r, 1)], idx_b[b],
                            ix_sem[b]).wait()
      pltpu.async_copy(x_hbm.at[idx_b[b].at[0, 0]], rows_b[b], g_sem[b])

    def consume(ci, b):
      r = row0 + ci
      pltpu.make_async_copy(ep_hbm.at[pl.ds(r * G, G)], ep_b[b],
                            ep_sem[b]).wait()
      pltpu.make_async_copy(x_hbm.at[idx_b[b].at[0, 0]], rows_b[b],
                            g_sem[b]).wait()

      def _crow(rr, inner):
        for k in range(D // LANE):
          sl = pl.ds(k * LANE, LANE)
          rows_b[b][rr, sl] = jnp.maximum(
              rows_b[b][rr, sl] + ep_b[b][rr, sl], 0.0)
        return inner
      lax.fori_loop(0, G, _crow, 0)
      pltpu.async_copy(rows_b[b], aggr_sh.at[idx_b[b].at[0, 1]], sc_sem[b],
                       add=True)

    def wait_scatter(b):
      pltpu.make_async_copy(rows_b[b], aggr_sh.at[idx_b[b].at[0, 1]],
                            sc_sem[b]).wait()

    stage1(0, 0)
    stage1(1, 1)
    stage2(0, 0)

    def _pair(t, carry):
      for j in range(NBUF):
        ci = t * NBUF + j

        @pl.when(ci + 1 < RPT)
        def _():
          stage2(ci + 1, (j + 1) % NBUF)

        consume(ci, j)
        wait_scatter(j)

        @pl.when(ci + NBUF < RPT)
        def _():
          stage1(ci + NBUF, j)
      return carry
    lax.fori_loop(0, TRIPS, _pair, 0)
    for ci in range(TRIPS * NBUF, RPT):
      consume(ci, ci % NBUF)
      wait_scatter(ci % NBUF)

    plsc.subcore_barrier()
    pltpu.sync_copy(aggr_sh.at[pl.ds(z0, NZR)],
                    out_hbm.at[c, pl.ds(z0, NZR)])

  return agg_kernel(x, idx_cat, ep)


def _edge_proj(edge_attr, We, be):
  BE = 2000

  def body(ea_ref, we_ref, be_ref, out_ref):
    out_ref[...] = (jnp.dot(ea_ref[...], we_ref[...],
                            preferred_element_type=jnp.float32) + be_ref[...])

  return pl.pallas_call(
      body,
      grid=(E // BE,),
      in_specs=[
          pl.BlockSpec((BE, ED), lambda i: (i, 0)),
          pl.BlockSpec((ED, D), lambda i: (0, 0)),
          pl.BlockSpec((1, D), lambda i: (0, 0)),
      ],
      out_specs=pl.BlockSpec((BE, D), lambda i: (i, 0)),
      out_shape=jax.ShapeDtypeStruct((E, D), jnp.float32),
  )(edge_attr, We, be.reshape(1, D))


def _node_update(x, aggr, Wp, bp):
  BN = 1000

  def body(x_ref, a_ref, w_ref, b_ref, out_ref):
    y = x_ref[...] + a_ref[0] + a_ref[1]
    out_ref[...] = jnp.maximum(
        jnp.dot(y, w_ref[...], preferred_element_type=jnp.float32)
        + b_ref[...], 0.0)

  return pl.pallas_call(
      body,
      grid=(N // BN,),
      in_specs=[
          pl.BlockSpec((BN, D), lambda i: (i, 0)),
          pl.BlockSpec((NC, BN, D), lambda i: (0, i, 0)),
          pl.BlockSpec((D, D), lambda i: (0, 0)),
          pl.BlockSpec((1, D), lambda i: (0, 0)),
      ],
      out_specs=pl.BlockSpec((BN, D), lambda i: (i, 0)),
      out_shape=jax.ShapeDtypeStruct((N, D), jnp.float32),
  )(x, aggr, Wp, bp.reshape(1, D))


def kernel(x, edge_index, edge_attr,
           We0, be0, W0, b0, g0, bt0,
           We1, be1, W1, b1, g1, bt1,
           We2, be2, W2, b2, g2, bt2):
  scale = 1.0 / math.sqrt(1.0 + BN_EPS)
  src_rows = edge_index[0].reshape(ROWS, G)
  dst_rows = edge_index[1].reshape(ROWS, G)
  idx_cat = jnp.stack([src_rows, dst_rows], axis=1)

  eps = [_edge_proj(edge_attr, We, be)
         for We, be in ((We0, be0), (We1, be1), (We2, be2))]

  h = x
  for ep, W, b, g, bt in (
      (eps[0], W0, b0, g0, bt0),
      (eps[1], W1, b1, g1, bt1),
      (eps[2], W2, b2, g2, bt2)):
    aggr = _sc_aggregate(h, idx_cat, ep)
    gs = g * scale
    h = _node_update(h, aggr, W * gs[None, :], b * gs + bt)
  return h


# speedup vs baseline: 2.5150x; 1.0285x over previous
import functools
import math

import jax
import jax.numpy as jnp
from jax import lax
from jax.experimental import pallas as pl
from jax.experimental.pallas import tpu as pltpu
from jax.experimental.pallas import tpu_sc as plsc

N = 10000
E = 320000
D = 128
ED = 16
BN_EPS = 1e-5

NC = 2
NS = 16
LANE = 16

G = 80
ROWS = E // G
RPS = ROWS // NC
RPT = RPS // NS
NZR = N // NS
NBUF = 2
TRIPS = RPT // NBUF


def _sc_aggregate(x, edge_index, ep):
  mesh = plsc.VectorSubcoreMesh(core_axis_name="c", subcore_axis_name="s")

  @functools.partial(
      pl.kernel,
      out_type=jax.ShapeDtypeStruct((NC, N, D), jnp.float32),
      mesh=mesh,
      compiler_params=pltpu.CompilerParams(use_tc_tiling_on_sc=False),
      scratch_types=[
          pltpu.VMEM_SHARED((N, D), jnp.float32),
          [pltpu.VMEM((2, G), jnp.int32) for _ in range(NBUF)],
          [pltpu.VMEM((G, D), jnp.float32) for _ in range(NBUF)],
          [pltpu.VMEM((G, D), jnp.float32) for _ in range(NBUF)],
          [pltpu.SemaphoreType.DMA for _ in range(NBUF)],
          [pltpu.SemaphoreType.DMA for _ in range(NBUF)],
          [pltpu.SemaphoreType.DMA for _ in range(NBUF)],
          [pltpu.SemaphoreType.DMA for _ in range(NBUF)],
      ],
  )
  def agg_kernel(x_hbm, idx_hbm, ep_hbm, out_hbm,
                 aggr_sh, idx_b, ep_b, rows_b, ix_sem, ep_sem, g_sem, sc_sem):
    c = lax.axis_index("c")
    s = lax.axis_index("s")
    row0 = c * RPS + s * RPT

    def _zrow(r, carry):
      for k in range(D // LANE):
        rows_b[0][r, pl.ds(k * LANE, LANE)] = jnp.zeros((LANE,), jnp.float32)
      return carry
    lax.fori_loop(0, G, _zrow, 0)
    z0 = s * NZR
    nfull = NZR // G
    for q in range(nfull):
      pltpu.sync_copy(rows_b[0], aggr_sh.at[pl.ds(z0 + q * G, G)])
    rem = NZR - nfull * G
    if rem:
      pltpu.sync_copy(rows_b[0].at[pl.ds(0, rem)],
                      aggr_sh.at[pl.ds(z0 + nfull * G, rem)])
    plsc.subcore_barrier()

    def stage1(ci, b):
      r = row0 + ci
      pltpu.async_copy(idx_hbm.at[pl.ds(0, 2), pl.ds(r * G, G)], idx_b[b],
                       ix_sem[b])
      pltpu.async_copy(ep_hbm.at[pl.ds(r * G, G)], ep_b[b], ep_sem[b])

    def stage2(ci, b):
      r = row0 + ci
      pltpu.make_async_copy(idx_hbm.at[pl.ds(0, 2), pl.ds(r * G, G)],
                            idx_b[b], ix_sem[b]).wait()
      pltpu.async_copy(x_hbm.at[idx_b[b].at[0]], rows_b[b], g_sem[b])

    def consume(ci, b):
      r = row0 + ci
      pltpu.make_async_copy(ep_hbm.at[pl.ds(r * G, G)], ep_b[b],
                            ep_sem[b]).wait()
      pltpu.make_async_copy(x_hbm.at[idx_b[b].at[0]], rows_b[b],
                            g_sem[b]).wait()

      def _crow(rr, inner):
        for k in range(D // LANE):
          sl = pl.ds(k * LANE, LANE)
          rows_b[b][rr, sl] = jnp.maximum(
              rows_b[b][rr, sl] + ep_b[b][rr, sl], 0.0)
        return inner
      lax.fori_loop(0, G, _crow, 0)
      pltpu.async_copy(rows_b[b], aggr_sh.at[idx_b[b].at[1]], sc_sem[b],
                       add=True)

    def wait_scatter(b):
      pltpu.make_async_copy(rows_b[b], aggr_sh.at[idx_b[b].at[1]],
                            sc_sem[b]).wait()

    stage1(0, 0)
    stage1(1, 1)
    stage2(0, 0)

    def _pair(t, carry):
      for j in range(NBUF):
        ci = t * NBUF + j

        @pl.when(ci + 1 < RPT)
        def _():
          stage2(ci + 1, (j + 1) % NBUF)

        consume(ci, j)
        wait_scatter(j)

        @pl.when(ci + NBUF < RPT)
        def _():
          stage1(ci + NBUF, j)
      return carry
    lax.fori_loop(0, TRIPS, _pair, 0)
    for ci in range(TRIPS * NBUF, RPT):
      consume(ci, ci % NBUF)
      wait_scatter(ci % NBUF)

    plsc.subcore_barrier()
    pltpu.sync_copy(aggr_sh.at[pl.ds(z0, NZR)],
                    out_hbm.at[c, pl.ds(z0, NZR)])

  return agg_kernel(x, edge_index, ep)


def _edge_proj(edge_attr, We, be):
  BE = 2000

  def body(ea_ref, we_ref, be_ref, out_ref):
    out_ref[...] = (jnp.dot(ea_ref[...], we_ref[...],
                            preferred_element_type=jnp.float32) + be_ref[...])

  return pl.pallas_call(
      body,
      grid=(E // BE,),
      in_specs=[
          pl.BlockSpec((BE, ED), lambda i: (i, 0)),
          pl.BlockSpec((ED, D), lambda i: (0, 0)),
          pl.BlockSpec((1, D), lambda i: (0, 0)),
      ],
      out_specs=pl.BlockSpec((BE, D), lambda i: (i, 0)),
      out_shape=jax.ShapeDtypeStruct((E, D), jnp.float32),
  )(edge_attr, We, be.reshape(1, D))


def _node_update(x, aggr, Wp, bp):
  BN = 1000

  def body(x_ref, a_ref, w_ref, b_ref, out_ref):
    y = x_ref[...] + a_ref[0] + a_ref[1]
    out_ref[...] = jnp.maximum(
        jnp.dot(y, w_ref[...], preferred_element_type=jnp.float32)
        + b_ref[...], 0.0)

  return pl.pallas_call(
      body,
      grid=(N // BN,),
      in_specs=[
          pl.BlockSpec((BN, D), lambda i: (i, 0)),
          pl.BlockSpec((NC, BN, D), lambda i: (0, i, 0)),
          pl.BlockSpec((D, D), lambda i: (0, 0)),
          pl.BlockSpec((1, D), lambda i: (0, 0)),
      ],
      out_specs=pl.BlockSpec((BN, D), lambda i: (i, 0)),
      out_shape=jax.ShapeDtypeStruct((N, D), jnp.float32),
  )(x, aggr, Wp, bp.reshape(1, D))


def kernel(x, edge_index, edge_attr,
           We0, be0, W0, b0, g0, bt0,
           We1, be1, W1, b1, g1, bt1,
           We2, be2, W2, b2, g2, bt2):
  scale = 1.0 / math.sqrt(1.0 + BN_EPS)

  eps = [_edge_proj(edge_attr, We, be)
         for We, be in ((We0, be0), (We1, be1), (We2, be2))]

  h = x
  for ep, W, b, g, bt in (
      (eps[0], W0, b0, g0, bt0),
      (eps[1], W1, b1, g1, bt1),
      (eps[2], W2, b2, g2, bt2)):
    aggr = _sc_aggregate(h, edge_index, ep)
    gs = g * scale
    h = _node_update(h, aggr, W * gs[None, :], b * gs + bt)
  return h


# speedup vs baseline: 3.0160x; 1.1992x over previous
import functools
import math

import jax
import jax.numpy as jnp
from jax import lax
from jax.experimental import pallas as pl
from jax.experimental.pallas import tpu as pltpu
from jax.experimental.pallas import tpu_sc as plsc

N = 10000
E = 320000
D = 128
ED = 16
BN_EPS = 1e-5

NC = 2
NS = 16
LANE = 16

G = 80
ROWS = E // G
RPS = ROWS // NC
RPT = RPS // NS
NZR = N // NS
NBUF = 2
TRIPS = RPT // NBUF


def _sc_aggregate(x, edge_index, ep):
  mesh = plsc.VectorSubcoreMesh(core_axis_name="c", subcore_axis_name="s")

  @functools.partial(
      pl.kernel,
      out_type=jax.ShapeDtypeStruct((NC, N, D), jnp.float32),
      mesh=mesh,
      compiler_params=pltpu.CompilerParams(use_tc_tiling_on_sc=False),
      scratch_types=[
          pltpu.VMEM_SHARED((N, D), jnp.float32),
          [pltpu.VMEM((2, G), jnp.int32) for _ in range(NBUF)],
          [pltpu.VMEM((G, D), jnp.float32) for _ in range(NBUF)],
          [pltpu.VMEM((G, D), jnp.float32) for _ in range(NBUF)],
          [pltpu.SemaphoreType.DMA for _ in range(NBUF)],
          [pltpu.SemaphoreType.DMA for _ in range(NBUF)],
          [pltpu.SemaphoreType.DMA for _ in range(NBUF)],
          [pltpu.SemaphoreType.DMA for _ in range(NBUF)],
      ],
  )
  def agg_kernel(x_hbm, idx_hbm, ep_hbm, out_hbm,
                 aggr_sh, idx_b, ep_b, rows_b, ix_sem, ep_sem, g_sem, sc_sem):
    c = lax.axis_index("c")
    s = lax.axis_index("s")
    row0 = c * RPS + s * RPT

    def _zrow(r, carry):
      for k in range(D // LANE):
        rows_b[0][r, pl.ds(k * LANE, LANE)] = jnp.zeros((LANE,), jnp.float32)
      return carry
    lax.fori_loop(0, G, _zrow, 0)
    z0 = s * NZR
    nfull = NZR // G
    for q in range(nfull):
      pltpu.sync_copy(rows_b[0], aggr_sh.at[pl.ds(z0 + q * G, G)])
    rem = NZR - nfull * G
    if rem:
      pltpu.sync_copy(rows_b[0].at[pl.ds(0, rem)],
                      aggr_sh.at[pl.ds(z0 + nfull * G, rem)])
    plsc.subcore_barrier()

    def stage1(ci, b):
      r = row0 + ci
      pltpu.async_copy(idx_hbm.at[pl.ds(0, 2), pl.ds(r * G, G)], idx_b[b],
                       ix_sem[b])
      pltpu.async_copy(ep_hbm.at[pl.ds(r * G, G)], ep_b[b], ep_sem[b])

    def stage2(ci, b):
      r = row0 + ci
      pltpu.make_async_copy(idx_hbm.at[pl.ds(0, 2), pl.ds(r * G, G)],
                            idx_b[b], ix_sem[b]).wait()
      pltpu.async_copy(x_hbm.at[idx_b[b].at[0]], rows_b[b], g_sem[b])

    def consume(ci, b):
      r = row0 + ci
      pltpu.make_async_copy(ep_hbm.at[pl.ds(r * G, G)], ep_b[b],
                            ep_sem[b]).wait()
      pltpu.make_async_copy(x_hbm.at[idx_b[b].at[0]], rows_b[b],
                            g_sem[b]).wait()

      def _crow(rr, inner):
        for k in range(D // LANE):
          sl = pl.ds(k * LANE, LANE)
          rows_b[b][rr, sl] = jnp.maximum(
              rows_b[b][rr, sl] + ep_b[b][rr, sl], 0.0)
        return inner
      lax.fori_loop(0, G, _crow, 0)
      pltpu.async_copy(rows_b[b], aggr_sh.at[idx_b[b].at[1]], sc_sem[b],
                       add=True)

    def wait_scatter(b):
      pltpu.make_async_copy(rows_b[b], aggr_sh.at[idx_b[b].at[1]],
                            sc_sem[b]).wait()

    stage1(0, 0)
    stage1(1, 1)
    stage2(0, 0)

    def _pair(t, carry):
      for j in range(NBUF):
        ci = t * NBUF + j

        @pl.when(ci + 1 < RPT)
        def _():
          stage2(ci + 1, (j + 1) % NBUF)

        consume(ci, j)
        wait_scatter(j)

        @pl.when(ci + NBUF < RPT)
        def _():
          stage1(ci + NBUF, j)
      return carry
    lax.fori_loop(0, TRIPS, _pair, 0)
    for ci in range(TRIPS * NBUF, RPT):
      consume(ci, ci % NBUF)
      wait_scatter(ci % NBUF)

    plsc.subcore_barrier()
    pltpu.sync_copy(aggr_sh.at[pl.ds(z0, NZR)],
                    out_hbm.at[c, pl.ds(z0, NZR)])

  return agg_kernel(x, edge_index, ep)


def _edge_proj(edge_attr_t, We, be):
  BE = 3200

  def body(ea_ref, we_ref, be_ref, out_ref):
    out_ref[...] = (lax.dot_general(
        ea_ref[...], we_ref[...],
        dimension_numbers=(((0,), (0,)), ((), ())),
        preferred_element_type=jnp.float32) + be_ref[...])

  return pl.pallas_call(
      body,
      grid=(E // BE,),
      in_specs=[
          pl.BlockSpec((ED, BE), lambda i: (0, i)),
          pl.BlockSpec((ED, D), lambda i: (0, 0)),
          pl.BlockSpec((1, D), lambda i: (0, 0)),
      ],
      out_specs=pl.BlockSpec((BE, D), lambda i: (i, 0)),
      out_shape=jax.ShapeDtypeStruct((E, D), jnp.float32),
  )(edge_attr_t, We, be.reshape(1, D))


def _node_update(x, aggr, Wp, bp):
  BN = 1000

  def body(x_ref, a_ref, w_ref, b_ref, out_ref):
    y = x_ref[...] + a_ref[0] + a_ref[1]
    out_ref[...] = jnp.maximum(
        jnp.dot(y, w_ref[...], preferred_element_type=jnp.float32)
        + b_ref[...], 0.0)

  return pl.pallas_call(
      body,
      grid=(N // BN,),
      in_specs=[
          pl.BlockSpec((BN, D), lambda i: (i, 0)),
          pl.BlockSpec((NC, BN, D), lambda i: (0, i, 0)),
          pl.BlockSpec((D, D), lambda i: (0, 0)),
          pl.BlockSpec((1, D), lambda i: (0, 0)),
      ],
      out_specs=pl.BlockSpec((BN, D), lambda i: (i, 0)),
      out_shape=jax.ShapeDtypeStruct((N, D), jnp.float32),
  )(x, aggr, Wp, bp.reshape(1, D))


def kernel(x, edge_index, edge_attr,
           We0, be0, W0, b0, g0, bt0,
           We1, be1, W1, b1, g1, bt1,
           We2, be2, W2, b2, g2, bt2):
  scale = 1.0 / math.sqrt(1.0 + BN_EPS)

  ea_t = edge_attr.T
  eps = [_edge_proj(ea_t, We, be)
         for We, be in ((We0, be0), (We1, be1), (We2, be2))]

  h = x
  for ep, W, b, g, bt in (
      (eps[0], W0, b0, g0, bt0),
      (eps[1], W1, b1, g1, bt1),
      (eps[2], W2, b2, g2, bt2)):
    aggr = _sc_aggregate(h, edge_index, ep)
    gs = g * scale
    h = _node_update(h, aggr, W * gs[None, :], b * gs + bt)
  return h
